# baseline (speedup 1.0000x reference)
import functools

import jax
import jax.numpy as jnp
from jax import lax
from jax.experimental import pallas as pl
from jax.experimental.pallas import tpu as pltpu
from jax.experimental.pallas import tpu_sc as plsc

_N = 10000
_E = 320000
_D = 128
_G = 64
_OUT = 64
_NPAD = 10112
_NC = 2
_NS = 16
_NW = _NC * _NS
_C = 128
_E2 = _E + _N
_Q = 2 * (-(-_E2 // (_NW * _C * 2)))
_E2P = _NW * _Q * _C
_SEG = 8
_NSEG = -(-_Q // _SEG)
_QP = _NSEG * _SEG
_RPT = _NPAD // _NS


def _zero_rows(ref, nrows, width):
    def row(i, _):
        for j in range(width // 16):
            ref[i, pl.ds(j * 16, 16)] = jnp.zeros((16,), jnp.float32)
        return 0
    lax.fori_loop(0, nrows, row, 0)


@functools.cache
def _sc_kernels():
    mesh = plsc.VectorSubcoreMesh(core_axis_name="c", subcore_axis_name="s",
                                  num_cores=_NC, num_subcores=_NS)

    @functools.partial(
        pl.kernel,
        out_type=jax.ShapeDtypeStruct((_NC * _NPAD, _D), jnp.float32),
        mesh=mesh,
        scratch_types=[
            pltpu.VMEM((_QP, _C), jnp.int32),
            pltpu.VMEM((_C, _D), jnp.float32),
            pltpu.VMEM((_C, _D), jnp.float32),
            pltpu.VMEM_SHARED((_NPAD, _D), jnp.float32),
        ],
    )
    def deg_kernel(dstw, degp, dst_v, ones_v, zbuf_v, acc):
        c = lax.axis_index("c")
        s = lax.axis_index("s")
        wid = s * _NC + c
        pltpu.sync_copy(dstw.at[pl.ds(wid * _QP, _QP)], dst_v)
        def ones_row(i, _):
            for j in range(_D // 16):
                ones_v[i, pl.ds(j * 16, 16)] = jnp.ones((16,), jnp.float32)
            return 0
        lax.fori_loop(0, _C, ones_row, 0)
        _zero_rows(zbuf_v, _C, _D)
        off = 0
        while off < _RPT:
            n = min(_C, _RPT - off)
            pltpu.sync_copy(zbuf_v.at[pl.ds(0, n)],
                            acc.at[pl.ds(s * _RPT + off, n)])
            off += n
        plsc.subcore_barrier()
        def step(q, _):
            pltpu.sync_copy(ones_v, acc.at[dst_v.at[q]], add=True)
            return 0
        lax.fori_loop(0, _Q, step, 0)
        plsc.subcore_barrier()
        pltpu.sync_copy(acc.at[pl.ds(s * _RPT, _RPT)],
                        degp.at[pl.ds(c * _NPAD + s * _RPT, _RPT)])

    @functools.partial(
        pl.kernel,
        out_type=jax.ShapeDtypeStruct((_NC * _NPAD, _D), jnp.float32),
        mesh=mesh,
        scratch_types=[
            pltpu.VMEM((_Q, _C), jnp.int32),
            pltpu.VMEM((_QP, _C), jnp.int32),
            pltpu.VMEM((_C, _D), jnp.float32),
            pltpu.VMEM_SHARED((_NPAD, _D), jnp.float32),
            pltpu.SemaphoreType.DMA,
        ],
    )
    def agg_kernel(g_hbm, srcw, dstw, outp, src_v, dst_v, rows0, acc, sem0):
        c = lax.axis_index("c")
        s = lax.axis_index("s")
        wid = s * _NC + c
        pltpu.sync_copy(srcw.at[wid], src_v)
        pltpu.sync_copy(dstw.at[pl.ds(wid * _QP, _QP)], dst_v)
        _zero_rows(rows0, _C, _D)
        off = 0
        while off < _RPT:
            n = min(_C, _RPT - off)
            pltpu.sync_copy(rows0.at[pl.ds(0, n)],
                            acc.at[pl.ds(s * _RPT + off, n)])
            off += n
        plsc.subcore_barrier()

        def gstart(q, rows, sem):
            pltpu.async_copy(g_hbm.at[src_v.at[q]], rows, sem)

        def gwait(q, rows, sem):
            pltpu.make_async_copy(g_hbm.at[src_v.at[q]], rows, sem).wait()

        def scat(q, rows):
            pltpu.sync_copy(rows, acc.at[dst_v.at[q]], add=True)

        def step(q, _):
            gstart(q, rows0, sem0)
            gwait(q, rows0, sem0)
            scat(q, rows0)
            return 0
        lax.fori_loop(0, _Q, step, 0)
        plsc.subcore_barrier()
        pltpu.sync_copy(acc.at[pl.ds(s * _RPT, _RPT)],
                        outp.at[pl.ds(c * _NPAD + s * _RPT, _RPT)])

    return deg_kernel, agg_kernel


def _tc_first(degp, x, w1):
    def body(degp_ref, x_ref, w_ref, dinv_ref, g_ref):
        deg = degp_ref[0, :, 0:1] + degp_ref[1, :, 0:1]
        dinv = jnp.where(deg > 0, lax.rsqrt(deg), 0.0)
        dinv_ref[...] = dinv
        h = jnp.dot(x_ref[...], w_ref[...], preferred_element_type=jnp.float32)
        g_ref[...] = h * dinv[:_N]
    return pl.pallas_call(
        body,
        out_shape=(jax.ShapeDtypeStruct((_NPAD, 1), jnp.float32),
                   jax.ShapeDtypeStruct((_N, _D), jnp.float32)),
    )(degp.reshape(_NC, _NPAD, _D), x, w1)


def _tc_mid(p, dinv, b, w):
    def body(p_ref, dinv_ref, b_ref, w_ref, g_ref):
        dinv = dinv_ref[...]
        agg = p_ref[0, :_N] + p_ref[1, :_N]
        a = jnp.maximum(agg * dinv[:_N] + b_ref[...], 0.0)
        g_ref[...] = jnp.dot(a, w_ref[...],
                             preferred_element_type=jnp.float32) * dinv[:_N]
    return pl.pallas_call(
        body,
        out_shape=jax.ShapeDtypeStruct((_N, _D), jnp.float32),
    )(p.reshape(_NC, _NPAD, _D), dinv, b.reshape(1, _D), w)


def _tc_head(p, dinv, b3, batch_pad, wl1, bl1, wl2, bl2):
    def body(p_ref, dinv_ref, b_ref, batch_ref, wl1_ref, bl1_ref, wl2_ref,
             bl2_ref, out_ref, emb_ref):
        dinv = dinv_ref[...]
        agg = p_ref[0] + p_ref[1]
        a = jnp.maximum(agg * dinv + b_ref[...], 0.0)
        gid = lax.broadcasted_iota(jnp.int32, (_G, _NPAD), 0)
        onehot = (batch_ref[...] == gid).astype(jnp.float32)
        sums = jnp.dot(onehot, a, preferred_element_type=jnp.float32)
        cnts = jnp.sum(onehot, axis=1, keepdims=True)
        pooled = sums / jnp.maximum(cnts, 1.0)
        emb = jnp.maximum(
            jnp.dot(pooled, wl1_ref[...], preferred_element_type=jnp.float32)
            + bl1_ref[...], 0.0)
        emb_ref[...] = emb
        out_ref[...] = (jnp.dot(emb, wl2_ref[...],
                                preferred_element_type=jnp.float32)
                        + bl2_ref[...])
    return pl.pallas_call(
        body,
        out_shape=(jax.ShapeDtypeStruct((_G, _OUT), jnp.float32),
                   jax.ShapeDtypeStruct((_G, _D), jnp.float32)),
    )(p.reshape(_NC, _NPAD, _D), dinv, b3.reshape(1, _D), batch_pad,
      wl1, bl1.reshape(1, _D), wl2, bl2.reshape(1, _OUT))


def kernel(x, edge_index, batch, W1, b1, W2, b2, W3, b3, Wl1, bl1, Wl2, bl2):
    deg_kernel, agg_kernel = _sc_kernels()

    loop = jnp.arange(_N, dtype=jnp.int32)
    src = jnp.concatenate(
        [edge_index[0], loop, jnp.zeros((_E2P - _E2,), jnp.int32)])
    junk = _N + jnp.arange(_E2P - _E2, dtype=jnp.int32) % (_NPAD - _N)
    dst = jnp.concatenate([edge_index[1], loop, junk])
    srcw = src.reshape(_NW, _Q, _C)
    dstw = jnp.pad(dst.reshape(_NW, _Q, _C), ((0, 0), (0, _QP - _Q), (0, 0)),
                   constant_values=_N).reshape(_NW * _QP, _C)
    batch_pad = jnp.concatenate(
        [batch, jnp.full((_NPAD - _N,), _G, jnp.int32)]).reshape(1, _NPAD)

    degp = deg_kernel(dstw)
    dinv, g = _tc_first(degp, x, W1)
    p = agg_kernel(g, srcw, dstw)
    g = _tc_mid(p, dinv, b1, W2)
    p = agg_kernel(g, srcw, dstw)
    g = _tc_mid(p, dinv, b2, W3)
    p = agg_kernel(g, srcw, dstw)
    out, emb = _tc_head(p, dinv, b3, batch_pad, Wl1, bl1, Wl2, bl2)
    return (out, emb)


# speedup vs baseline: 1.0004x; 1.0004x over previous
import functools

import jax
import jax.numpy as jnp
from jax import lax
from jax.experimental import pallas as pl
from jax.experimental.pallas import tpu as pltpu
from jax.experimental.pallas import tpu_sc as plsc

_N = 10000
_E = 320000
_D = 128
_G = 64
_OUT = 64
_NPAD = 10112
_NC = 2
_NS = 16
_NW = _NC * _NS
_C = 128
_E2 = _E + _N
_Q = 2 * (-(-_E2 // (_NW * _C * 2)))
_E2P = _NW * _Q * _C
_SEG = 8
_NSEG = -(-_Q // _SEG)
_QP = _NSEG * _SEG
_RPT = _NPAD // _NS


def _zero_rows(ref, nrows, width):
    def row(i, _):
        for j in range(width // 16):
            ref[i, pl.ds(j * 16, 16)] = jnp.zeros((16,), jnp.float32)
        return 0
    lax.fori_loop(0, nrows, row, 0)


@functools.cache
def _sc_kernels():
    mesh = plsc.VectorSubcoreMesh(core_axis_name="c", subcore_axis_name="s",
                                  num_cores=_NC, num_subcores=_NS)

    @functools.partial(
        pl.kernel,
        out_type=jax.ShapeDtypeStruct((_NC * _NPAD, _D), jnp.float32),
        mesh=mesh,
        scratch_types=[
            pltpu.VMEM((_QP, _C), jnp.int32),
            pltpu.VMEM((_C, _D), jnp.float32),
            pltpu.VMEM((_C, _D), jnp.float32),
            pltpu.VMEM_SHARED((_NPAD, _D), jnp.float32),
        ],
    )
    def deg_kernel(dstw, degp, dst_v, ones_v, zbuf_v, acc):
        c = lax.axis_index("c")
        s = lax.axis_index("s")
        wid = s * _NC + c
        pltpu.sync_copy(dstw.at[pl.ds(wid * _QP, _QP)], dst_v)
        def ones_row(i, _):
            for j in range(_D // 16):
                ones_v[i, pl.ds(j * 16, 16)] = jnp.ones((16,), jnp.float32)
            return 0
        lax.fori_loop(0, _C, ones_row, 0)
        _zero_rows(zbuf_v, _C, _D)
        off = 0
        while off < _RPT:
            n = min(_C, _RPT - off)
            pltpu.sync_copy(zbuf_v.at[pl.ds(0, n)],
                            acc.at[pl.ds(s * _RPT + off, n)])
            off += n
        plsc.subcore_barrier()
        def step(q, _):
            pltpu.sync_copy(ones_v, acc.at[dst_v.at[q]], add=True)
            return 0
        lax.fori_loop(0, _Q, step, 0)
        plsc.subcore_barrier()
        pltpu.sync_copy(acc.at[pl.ds(s * _RPT, _RPT)],
                        degp.at[pl.ds(c * _NPAD + s * _RPT, _RPT)])

    @functools.partial(
        pl.kernel,
        out_type=jax.ShapeDtypeStruct((_NC * _NPAD, _D), jnp.float32),
        mesh=mesh,
        scratch_types=[
            pltpu.VMEM((_Q, _C), jnp.int32),
            pltpu.VMEM((_QP, _C), jnp.int32),
            pltpu.VMEM((_C, _D), jnp.float32),
            pltpu.VMEM_SHARED((_NPAD, _D), jnp.float32),
            pltpu.SemaphoreType.DMA,
        ],
    )
    def agg_kernel(g_hbm, srcw, dstw, outp, src_v, dst_v, rows0, acc, sem0):
        c = lax.axis_index("c")
        s = lax.axis_index("s")
        wid = s * _NC + c
        pltpu.sync_copy(srcw.at[wid], src_v)
        pltpu.sync_copy(dstw.at[pl.ds(wid * _QP, _QP)], dst_v)
        _zero_rows(rows0, _C, _D)
        off = 0
        while off < _RPT:
            n = min(_C, _RPT - off)
            pltpu.sync_copy(rows0.at[pl.ds(0, n)],
                            acc.at[pl.ds(s * _RPT + off, n)])
            off += n
        plsc.subcore_barrier()

        def gstart(q, rows, sem):
            pltpu.async_copy(g_hbm.at[src_v.at[q]], rows, sem)

        def gwait(q, rows, sem):
            pltpu.make_async_copy(g_hbm.at[src_v.at[q]], rows, sem).wait()

        def scat(q, rows):
            pltpu.sync_copy(rows, acc.at[dst_v.at[q]], add=True)

        def step(q, _):
            pltpu.async_copy(g_hbm.at[src_v.at[q]], rows0, sem0).wait()
            pltpu.sync_copy(rows0, acc.at[dst_v.at[q]], add=True)
            return 0
        lax.fori_loop(0, _Q, step, 0)
        plsc.subcore_barrier()
        pltpu.sync_copy(acc.at[pl.ds(s * _RPT, _RPT)],
                        outp.at[pl.ds(c * _NPAD + s * _RPT, _RPT)])

    return deg_kernel, agg_kernel


def _tc_first(degp, x, w1):
    def body(degp_ref, x_ref, w_ref, dinv_ref, g_ref):
        deg = degp_ref[0, :, 0:1] + degp_ref[1, :, 0:1]
        dinv = jnp.where(deg > 0, lax.rsqrt(deg), 0.0)
        dinv_ref[...] = dinv
        h = jnp.dot(x_ref[...], w_ref[...], preferred_element_type=jnp.float32)
        g_ref[...] = h * dinv[:_N]
    return pl.pallas_call(
        body,
        out_shape=(jax.ShapeDtypeStruct((_NPAD, 1), jnp.float32),
                   jax.ShapeDtypeStruct((_N, _D), jnp.float32)),
    )(degp.reshape(_NC, _NPAD, _D), x, w1)


def _tc_mid(p, dinv, b, w):
    def body(p_ref, dinv_ref, b_ref, w_ref, g_ref):
        dinv = dinv_ref[...]
        agg = p_ref[0, :_N] + p_ref[1, :_N]
        a = jnp.maximum(agg * dinv[:_N] + b_ref[...], 0.0)
        g_ref[...] = jnp.dot(a, w_ref[...],
                             preferred_element_type=jnp.float32) * dinv[:_N]
    return pl.pallas_call(
        body,
        out_shape=jax.ShapeDtypeStruct((_N, _D), jnp.float32),
    )(p.reshape(_NC, _NPAD, _D), dinv, b.reshape(1, _D), w)


def _tc_head(p, dinv, b3, batch_pad, wl1, bl1, wl2, bl2):
    def body(p_ref, dinv_ref, b_ref, batch_ref, wl1_ref, bl1_ref, wl2_ref,
             bl2_ref, out_ref, emb_ref):
        dinv = dinv_ref[...]
        agg = p_ref[0] + p_ref[1]
        a = jnp.maximum(agg * dinv + b_ref[...], 0.0)
        gid = lax.broadcasted_iota(jnp.int32, (_G, _NPAD), 0)
        onehot = (batch_ref[...] == gid).astype(jnp.float32)
        sums = jnp.dot(onehot, a, preferred_element_type=jnp.float32)
        cnts = jnp.sum(onehot, axis=1, keepdims=True)
        pooled = sums / jnp.maximum(cnts, 1.0)
        emb = jnp.maximum(
            jnp.dot(pooled, wl1_ref[...], preferred_element_type=jnp.float32)
            + bl1_ref[...], 0.0)
        emb_ref[...] = emb
        out_ref[...] = (jnp.dot(emb, wl2_ref[...],
                                preferred_element_type=jnp.float32)
                        + bl2_ref[...])
    return pl.pallas_call(
        body,
        out_shape=(jax.ShapeDtypeStruct((_G, _OUT), jnp.float32),
                   jax.ShapeDtypeStruct((_G, _D), jnp.float32)),
    )(p.reshape(_NC, _NPAD, _D), dinv, b3.reshape(1, _D), batch_pad,
      wl1, bl1.reshape(1, _D), wl2, bl2.reshape(1, _OUT))


def kernel(x, edge_index, batch, W1, b1, W2, b2, W3, b3, Wl1, bl1, Wl2, bl2):
    deg_kernel, agg_kernel = _sc_kernels()

    loop = jnp.arange(_N, dtype=jnp.int32)
    src = jnp.concatenate(
        [edge_index[0], loop, jnp.zeros((_E2P - _E2,), jnp.int32)])
    junk = _N + jnp.arange(_E2P - _E2, dtype=jnp.int32) % (_NPAD - _N)
    dst = jnp.concatenate([edge_index[1], loop, junk])
    srcw = src.reshape(_NW, _Q, _C)
    dstw = jnp.pad(dst.reshape(_NW, _Q, _C), ((0, 0), (0, _QP - _Q), (0, 0)),
                   constant_values=_N).reshape(_NW * _QP, _C)
    batch_pad = jnp.concatenate(
        [batch, jnp.full((_NPAD - _N,), _G, jnp.int32)]).reshape(1, _NPAD)

    degp = deg_kernel(dstw)
    dinv, g = _tc_first(degp, x, W1)
    p = agg_kernel(g, srcw, dstw)
    g = _tc_mid(p, dinv, b1, W2)
    p = agg_kernel(g, srcw, dstw)
    g = _tc_mid(p, dinv, b2, W3)
    p = agg_kernel(g, srcw, dstw)
    out, emb = _tc_head(p, dinv, b3, batch_pad, Wl1, bl1, Wl2, bl2)
    return (out, emb)


# speedup vs baseline: 1.9066x; 1.9059x over previous
import functools

import jax
import jax.numpy as jnp
from jax import lax
from jax.experimental import pallas as pl
from jax.experimental.pallas import tpu as pltpu
from jax.experimental.pallas import tpu_sc as plsc

_N = 10000
_E = 320000
_D = 128
_G = 64
_OUT = 64
_NPAD = 10240
_NC = 2
_NS = 16
_NW = _NC * _NS
_C = 128
_E2 = _E + _N
_Q = -(-_E2 // (_NW * _C))
_E2P = _NW * _Q * _C
_SEG = 8
_NSEG = -(-_Q // _SEG)
_QP = _NSEG * _SEG
_RPT = _NPAD // _NS


def _zero_rows(ref, nrows, width):
    def row(i, _):
        for j in range(width // 16):
            ref[i, pl.ds(j * 16, 16)] = jnp.zeros((16,), jnp.float32)
        return 0
    lax.fori_loop(0, nrows, row, 0)


@functools.cache
def _sc_kernels():
    mesh = plsc.VectorSubcoreMesh(core_axis_name="c", subcore_axis_name="s",
                                  num_cores=_NC, num_subcores=_NS)

    @functools.partial(
        pl.kernel,
        out_type=jax.ShapeDtypeStruct((_NC * _NPAD, _D), jnp.float32),
        mesh=mesh,
        scratch_types=[
            pltpu.VMEM((_Q, _C), jnp.int32),
            pltpu.VMEM((_C, _D), jnp.float32),
            pltpu.VMEM((_C, _D), jnp.float32),
            pltpu.VMEM_SHARED((_NPAD, _D), jnp.float32),
        ],
    )
    def deg_kernel(dstw, degp, dst_v, ones_v, zbuf_v, acc):
        c = lax.axis_index("c")
        s = lax.axis_index("s")
        wid = s * _NC + c
        pltpu.sync_copy(dstw.at[wid], dst_v)
        def ones_row(i, _):
            for j in range(_D // 16):
                ones_v[i, pl.ds(j * 16, 16)] = jnp.ones((16,), jnp.float32)
            return 0
        lax.fori_loop(0, _C, ones_row, 0)
        _zero_rows(zbuf_v, _C, _D)
        off = 0
        while off < _RPT:
            n = min(_C, _RPT - off)
            pltpu.sync_copy(zbuf_v.at[pl.ds(0, n)],
                            acc.at[pl.ds(s * _RPT + off, n)])
            off += n
        plsc.subcore_barrier()
        def step(q, _):
            pltpu.sync_copy(ones_v, acc.at[dst_v.at[q]], add=True)
            return 0
        lax.fori_loop(0, _Q, step, 0)
        plsc.subcore_barrier()
        pltpu.sync_copy(acc.at[pl.ds(s * _RPT, _RPT)],
                        degp.at[pl.ds(c * _NPAD + s * _RPT, _RPT)])

    @functools.partial(
        pl.kernel,
        out_type=jax.ShapeDtypeStruct((_NC * _NPAD, _D), jnp.float32),
        mesh=mesh,
        scratch_types=[
            pltpu.VMEM((_Q, _C), jnp.int32),
            pltpu.VMEM((_Q, _C), jnp.int32),
            pltpu.VMEM((_C, _D), jnp.float32),
            pltpu.VMEM_SHARED((_NPAD, _D), jnp.float32),
            pltpu.SemaphoreType.DMA,
        ],
    )
    def agg_kernel(g_hbm, srcw, dstw, outp, src_v, dst_v, rows0, acc, sem0):
        c = lax.axis_index("c")
        s = lax.axis_index("s")
        wid = s * _NC + c
        pltpu.sync_copy(srcw.at[wid], src_v)
        pltpu.sync_copy(dstw.at[wid], dst_v)
        _zero_rows(rows0, _C, _D)
        off = 0
        while off < _RPT:
            n = min(_C, _RPT - off)
            pltpu.sync_copy(rows0.at[pl.ds(0, n)],
                            acc.at[pl.ds(s * _RPT + off, n)])
            off += n
        plsc.subcore_barrier()

        def gstart(q, rows, sem):
            pltpu.async_copy(g_hbm.at[src_v.at[q]], rows, sem)

        def gwait(q, rows, sem):
            pltpu.make_async_copy(g_hbm.at[src_v.at[q]], rows, sem).wait()

        def scat(q, rows):
            pltpu.sync_copy(rows, acc.at[dst_v.at[q]], add=True)

        def step(q, _):
            pltpu.async_copy(g_hbm.at[src_v.at[q]], rows0, sem0).wait()
            pltpu.sync_copy(rows0, acc.at[dst_v.at[q]], add=True)
            return 0
        lax.fori_loop(0, _Q, step, 0)
        plsc.subcore_barrier()
        pltpu.sync_copy(acc.at[pl.ds(s * _RPT, _RPT)],
                        outp.at[pl.ds(c * _NPAD + s * _RPT, _RPT)])

    return deg_kernel, agg_kernel


def _tc_first(degp, x, w1):
    def body(degp_ref, x_ref, w_ref, dinv_ref, g_ref):
        deg = degp_ref[0, :, 0:1] + degp_ref[1, :, 0:1]
        dinv = jnp.where(deg > 0, lax.rsqrt(deg), 0.0)
        dinv_ref[...] = dinv
        h = jnp.dot(x_ref[...], w_ref[...], preferred_element_type=jnp.float32)
        g_ref[...] = h * dinv[:_N]
    return pl.pallas_call(
        body,
        out_shape=(jax.ShapeDtypeStruct((_NPAD, 1), jnp.float32),
                   jax.ShapeDtypeStruct((_N, _D), jnp.float32)),
    )(degp.reshape(_NC, _NPAD, _D), x, w1)


def _tc_mid(p, dinv, b, w):
    def body(p_ref, dinv_ref, b_ref, w_ref, g_ref):
        dinv = dinv_ref[...]
        agg = p_ref[0, :_N] + p_ref[1, :_N]
        a = jnp.maximum(agg * dinv[:_N] + b_ref[...], 0.0)
        g_ref[...] = jnp.dot(a, w_ref[...],
                             preferred_element_type=jnp.float32) * dinv[:_N]
    return pl.pallas_call(
        body,
        out_shape=jax.ShapeDtypeStruct((_N, _D), jnp.float32),
    )(p.reshape(_NC, _NPAD, _D), dinv, b.reshape(1, _D), w)


def _tc_head(p, dinv, b3, batch_pad, wl1, bl1, wl2, bl2):
    def body(p_ref, dinv_ref, b_ref, batch_ref, wl1_ref, bl1_ref, wl2_ref,
             bl2_ref, out_ref, emb_ref):
        dinv = dinv_ref[...]
        agg = p_ref[0] + p_ref[1]
        a = jnp.maximum(agg * dinv + b_ref[...], 0.0)
        gid = lax.broadcasted_iota(jnp.int32, (_G, _NPAD), 0)
        onehot = (batch_ref[...] == gid).astype(jnp.float32)
        sums = jnp.dot(onehot, a, preferred_element_type=jnp.float32)
        cnts = jnp.sum(onehot, axis=1, keepdims=True)
        pooled = sums / jnp.maximum(cnts, 1.0)
        emb = jnp.maximum(
            jnp.dot(pooled, wl1_ref[...], preferred_element_type=jnp.float32)
            + bl1_ref[...], 0.0)
        emb_ref[...] = emb
        out_ref[...] = (jnp.dot(emb, wl2_ref[...],
                                preferred_element_type=jnp.float32)
                        + bl2_ref[...])
    return pl.pallas_call(
        body,
        out_shape=(jax.ShapeDtypeStruct((_G, _OUT), jnp.float32),
                   jax.ShapeDtypeStruct((_G, _D), jnp.float32)),
    )(p.reshape(_NC, _NPAD, _D), dinv, b3.reshape(1, _D), batch_pad,
      wl1, bl1.reshape(1, _D), wl2, bl2.reshape(1, _OUT))


def kernel(x, edge_index, batch, W1, b1, W2, b2, W3, b3, Wl1, bl1, Wl2, bl2):
    deg_kernel, agg_kernel = _sc_kernels()

    loop = jnp.arange(_N, dtype=jnp.int32)
    src = jnp.concatenate(
        [edge_index[0], loop, jnp.zeros((_E2P - _E2,), jnp.int32)])
    junk = _N + jnp.arange(_E2P - _E2, dtype=jnp.int32) % (_NPAD - _N)
    dst = jnp.concatenate([edge_index[1], loop, junk])
    srcw = src.reshape(_NW, _Q, _C)
    dstw = dst.reshape(_NW, _Q, _C)
    batch_pad = jnp.concatenate(
        [batch, jnp.full((_NPAD - _N,), _G, jnp.int32)]).reshape(1, _NPAD)

    degp = deg_kernel(dstw)
    dinv, g = _tc_first(degp, x, W1)
    p = agg_kernel(g, srcw, dstw)
    g = _tc_mid(p, dinv, b1, W2)
    p = agg_kernel(g, srcw, dstw)
    g = _tc_mid(p, dinv, b2, W3)
    p = agg_kernel(g, srcw, dstw)
    out, emb = _tc_head(p, dinv, b3, batch_pad, Wl1, bl1, Wl2, bl2)
    return (out, emb)
